# baseline (speedup 1.0000x reference)
import functools

import jax
import jax.numpy as jnp
from jax.experimental import pallas as pl
from jax.experimental.pallas import tpu as pltpu

_EPS = 1e-5


def _fold_batchnorm(w_t, gamma, beta, mean, var):
    scale = gamma / jnp.sqrt(var + _EPS)
    return w_t * scale[None, :], (beta - mean * scale)[None, :]


def _bq_group_kernel(xyz_ref, feats_ref, xyzt_ref, cen_ref, tri_ref,
                     w1f_ref, b1_ref, wx_ref, w2_ref, b2_ref,
                     out_ref, *, radius2, nsample):
    xyzb = xyz_ref[0]
    feats = feats_ref[0]
    xyzt = xyzt_ref[0]
    cen = cen_ref[0]
    tri = tri_ref[...]

    tm = cen.shape[0]
    tn = feats.shape[0]
    c0 = w1f_ref.shape[1]
    sub = tri.shape[0]

    d0 = cen[:, 0:1] - xyzt[0:1, :]
    dist2 = d0 * d0
    for d in (1, 2):
        dd = cen[:, d:d + 1] - xyzt[d:d + 1, :]
        dist2 = dist2 + dd * dd
    in_ball = (dist2 < radius2).astype(jnp.float32)

    carry = jnp.zeros((tm, 1), jnp.float32)
    parts = []
    for s in range(tn // sub):
        ib = in_ball[:, s * sub:(s + 1) * sub].astype(jnp.bfloat16)
        lr = jnp.dot(ib, tri, preferred_element_type=jnp.float32)
        parts.append(lr + carry)
        carry = carry + lr[:, sub - 1:sub]
    rank = parts[0] if len(parts) == 1 else jnp.concatenate(parts, axis=1)
    total = carry

    rank_in = jnp.minimum(rank * in_ball,
                          float(2 * nsample)).astype(jnp.int8)
    kvec = (jax.lax.broadcasted_iota(jnp.int32, (nsample, 1, 1), 0)
            + 1).astype(jnp.int8)
    onehot = (rank_in[None, :, :] == kvec).astype(jnp.bfloat16)
    onehot2d = onehot.reshape(nsample * tm, tn)

    a = (jnp.dot(feats, w1f_ref[...], preferred_element_type=jnp.float32)
         + jnp.dot(xyzb, wx_ref[...], preferred_element_type=jnp.float32)
         + b1_ref[...])

    g = jnp.dot(onehot2d, a.astype(jnp.bfloat16),
                preferred_element_type=jnp.float32)

    bm = jnp.dot(cen, wx_ref[...],
                 preferred_element_type=jnp.float32)
    h1 = jax.nn.relu(g.reshape(nsample, tm, c0) - bm[None, :, :])
    h2 = jnp.dot(h1.reshape(nsample * tm, c0), w2_ref[...],
                 preferred_element_type=jnp.float32) + b2_ref[...]
    c1 = h2.shape[-1]
    h2 = jax.nn.relu(h2).reshape(nsample, tm, c1)
    kv = jax.lax.broadcasted_iota(jnp.int32, (nsample, 1, 1), 0) + 1
    valid = (kv <= total.astype(jnp.int32)[None, :, :])
    h2 = h2 * valid.astype(jnp.float32)
    out_ref[0] = jnp.max(h2, axis=0)


def _ball_group(xyz, new_xyz, feats, params, *, radius, nsample, tm,
                sub=128):
    B, N, _ = xyz.shape
    M = new_xyz.shape[1]
    Ci = feats.shape[-1]

    w1, s1 = _fold_batchnorm(params["w1"].T, params["g1"], params["b1"],
                             params["m1"], params["v1"])
    w2, s2 = _fold_batchnorm(params["w2"].T, params["g2"], params["b2"],
                             params["m2"], params["v2"])
    wx = w1[:3]
    C0, C1 = w1.shape[1], w2.shape[1]

    Mp = -(-M // tm) * tm
    Np = -(-N // sub) * sub
    FAR = 1e4
    xyz_p = jnp.pad(xyz, ((0, 0), (0, Np - N), (0, 0)), constant_values=FAR)
    feats_p = jnp.pad(feats, ((0, 0), (0, Np - N), (0, 0)))
    cen_p = jnp.pad(new_xyz, ((0, 0), (0, Mp - M), (0, 0)))

    xyzt = jnp.transpose(xyz_p, (0, 2, 1))
    tri = jnp.triu(jnp.ones((sub, sub), jnp.bfloat16))
    w1f = w1[3:]

    body = functools.partial(_bq_group_kernel,
                             radius2=float(radius) ** 2, nsample=int(nsample))
    out = pl.pallas_call(
        body,
        out_shape=jax.ShapeDtypeStruct((B, Mp, C1), jnp.float32),
        grid=(B, Mp // tm),
        in_specs=[
            pl.BlockSpec((1, Np, 3), lambda b, mi: (b, 0, 0)),
            pl.BlockSpec((1, Np, Ci), lambda b, mi: (b, 0, 0)),
            pl.BlockSpec((1, 3, Np), lambda b, mi: (b, 0, 0)),
            pl.BlockSpec((1, tm, 3), lambda b, mi: (b, mi, 0)),
            pl.BlockSpec((sub, sub), lambda b, mi: (0, 0)),
            pl.BlockSpec((Ci, C0), lambda b, mi: (0, 0)),
            pl.BlockSpec((1, C0), lambda b, mi: (0, 0)),
            pl.BlockSpec((3, C0), lambda b, mi: (0, 0)),
            pl.BlockSpec((C0, C1), lambda b, mi: (0, 0)),
            pl.BlockSpec((1, C1), lambda b, mi: (0, 0)),
        ],
        out_specs=pl.BlockSpec((1, tm, C1), lambda b, mi: (b, mi, 0)),
        compiler_params=pltpu.CompilerParams(
            dimension_semantics=("parallel", "parallel"),
            vmem_limit_bytes=100 * 1024 * 1024),
    )(xyz_p, feats_p, xyzt, cen_p, tri, w1f, s1, wx, w2, s2)
    return out[:, :M, :]


def kernel(xyz, new_xyz, feats, w1, w2, g1, b1, m1, v1, g2, b2, m2, v2):
    p = dict(w1=w1, w2=w2, g1=g1, b1=b1, m1=m1, v1=v1,
             g2=g2, b2=b2, m2=m2, v2=v2)
    return _ball_group(xyz, new_xyz, feats, p,
                       radius=0.2, nsample=32, tm=512, sub=128)


# speedup vs baseline: 1.3914x; 1.3914x over previous
import functools

import jax
import jax.numpy as jnp
from jax.experimental import pallas as pl
from jax.experimental.pallas import tpu as pltpu

_EPS = 1e-5


def _fold_batchnorm(w_t, gamma, beta, mean, var):
    scale = gamma / jnp.sqrt(var + _EPS)
    return w_t * scale[None, :], (beta - mean * scale)[None, :]


def _bq_group_kernel(xyz_ref, feats_ref, xyzt_ref, cen_ref, tri_ref,
                     w1f_ref, b1_ref, wx_ref, w2_ref, b2_ref,
                     out_ref, *, radius2, nsample):
    xyzb = xyz_ref[0]
    feats = feats_ref[0]
    xyzt = xyzt_ref[0]
    cen = cen_ref[0]
    tri = tri_ref[...]

    tm = cen.shape[0]
    tn = feats.shape[0]
    c0 = w1f_ref.shape[1]
    sub = tri.shape[0]

    d0 = cen[:, 0:1] - xyzt[0:1, :]
    dist2 = d0 * d0
    for d in (1, 2):
        dd = cen[:, d:d + 1] - xyzt[d:d + 1, :]
        dist2 = dist2 + dd * dd
    in_ball = (dist2 < radius2).astype(jnp.float32)

    carry = jnp.zeros((tm, 1), jnp.float32)
    parts = []
    for s in range(tn // sub):
        ib = in_ball[:, s * sub:(s + 1) * sub].astype(jnp.bfloat16)
        lr = jnp.dot(ib, tri, preferred_element_type=jnp.float32)
        parts.append(lr + carry)
        carry = carry + lr[:, sub - 1:sub]
    rank = parts[0] if len(parts) == 1 else jnp.concatenate(parts, axis=1)
    total = carry

    rank_in = (rank * in_ball).astype(jnp.int32)
    kvec = jax.lax.broadcasted_iota(jnp.int32, (nsample, 1, 1), 0) + 1
    onehot = (rank_in[None, :, :] == kvec).astype(jnp.bfloat16)
    onehot2d = onehot.reshape(nsample * tm, tn)

    a = (jnp.dot(feats, w1f_ref[...], preferred_element_type=jnp.float32)
         + jnp.dot(xyzb, wx_ref[...], preferred_element_type=jnp.float32)
         + b1_ref[...])

    g = jnp.dot(onehot2d, a.astype(jnp.bfloat16),
                preferred_element_type=jnp.float32)

    bm = jnp.dot(cen, wx_ref[...],
                 preferred_element_type=jnp.float32)
    h1 = jax.nn.relu(g.reshape(nsample, tm, c0) - bm[None, :, :])
    h2 = jnp.dot(h1.reshape(nsample * tm, c0), w2_ref[...],
                 preferred_element_type=jnp.float32) + b2_ref[...]
    c1 = h2.shape[-1]
    h2 = jax.nn.relu(h2).reshape(nsample, tm, c1)
    kv = jax.lax.broadcasted_iota(jnp.int32, (nsample, 1, 1), 0) + 1
    valid = (kv <= total.astype(jnp.int32)[None, :, :])
    h2 = h2 * valid.astype(jnp.float32)
    out_ref[0] = jnp.max(h2, axis=0)


def _ball_group(xyz, new_xyz, feats, params, *, radius, nsample, tm,
                sub=128):
    B, N, _ = xyz.shape
    M = new_xyz.shape[1]
    Ci = feats.shape[-1]

    w1, s1 = _fold_batchnorm(params["w1"].T, params["g1"], params["b1"],
                             params["m1"], params["v1"])
    w2, s2 = _fold_batchnorm(params["w2"].T, params["g2"], params["b2"],
                             params["m2"], params["v2"])
    wx = w1[:3]
    C0, C1 = w1.shape[1], w2.shape[1]

    Mp = -(-M // tm) * tm
    Np = -(-N // sub) * sub
    FAR = 1e4
    xyz_p = jnp.pad(xyz, ((0, 0), (0, Np - N), (0, 0)), constant_values=FAR)
    feats_p = jnp.pad(feats, ((0, 0), (0, Np - N), (0, 0)))
    cen_p = jnp.pad(new_xyz, ((0, 0), (0, Mp - M), (0, 0)))

    xyzt = jnp.transpose(xyz_p, (0, 2, 1))
    tri = jnp.triu(jnp.ones((sub, sub), jnp.bfloat16))
    w1f = w1[3:]

    body = functools.partial(_bq_group_kernel,
                             radius2=float(radius) ** 2, nsample=int(nsample))
    out = pl.pallas_call(
        body,
        out_shape=jax.ShapeDtypeStruct((B, Mp, C1), jnp.float32),
        grid=(B, Mp // tm),
        in_specs=[
            pl.BlockSpec((1, Np, 3), lambda b, mi: (b, 0, 0)),
            pl.BlockSpec((1, Np, Ci), lambda b, mi: (b, 0, 0)),
            pl.BlockSpec((1, 3, Np), lambda b, mi: (b, 0, 0)),
            pl.BlockSpec((1, tm, 3), lambda b, mi: (b, mi, 0)),
            pl.BlockSpec((sub, sub), lambda b, mi: (0, 0)),
            pl.BlockSpec((Ci, C0), lambda b, mi: (0, 0)),
            pl.BlockSpec((1, C0), lambda b, mi: (0, 0)),
            pl.BlockSpec((3, C0), lambda b, mi: (0, 0)),
            pl.BlockSpec((C0, C1), lambda b, mi: (0, 0)),
            pl.BlockSpec((1, C1), lambda b, mi: (0, 0)),
        ],
        out_specs=pl.BlockSpec((1, tm, C1), lambda b, mi: (b, mi, 0)),
        compiler_params=pltpu.CompilerParams(
            dimension_semantics=("parallel", "parallel"),
            vmem_limit_bytes=100 * 1024 * 1024),
    )(xyz_p, feats_p, xyzt, cen_p, tri, w1f, s1, wx, w2, s2)
    return out[:, :M, :]


def kernel(xyz, new_xyz, feats, w1, w2, g1, b1, m1, v1, g2, b2, m2, v2):
    p = dict(w1=w1, w2=w2, g1=g1, b1=b1, m1=m1, v1=v1,
             g2=g2, b2=b2, m2=m2, v2=v2)
    return _ball_group(xyz, new_xyz, feats, p,
                       radius=0.2, nsample=32, tm=512, sub=128)
